# baseline (speedup 1.0000x reference)
import jax
import jax.numpy as jnp
from jax.experimental import pallas as pl


def _scatter_mean(src_feats, src_idx, dst_idx, dim_size):
    msgs = jnp.take(src_feats, src_idx, axis=0)
    summed = jax.ops.segment_sum(msgs, dst_idx, num_segments=dim_size)
    counts = jax.ops.segment_sum(
        jnp.ones((dst_idx.shape[0],), dtype=src_feats.dtype), dst_idx, num_segments=dim_size
    )
    return summed / jnp.clip(counts, 1.0, None)[:, None]


def _combine_kernel(a_ref, b_ref, c_ref, d_ref, o_ref):
    o_ref[...] = 0.25 * (a_ref[...] + b_ref[...] + c_ref[...] + d_ref[...])


def _combine(a, b, c, d):
    return pl.pallas_call(
        _combine_kernel,
        out_shape=jax.ShapeDtypeStruct(a.shape, a.dtype),
        grid=(a.shape[0] // 1000,),
        in_specs=[pl.BlockSpec((1000, 128), lambda i: (i, 0))] * 4,
        out_specs=pl.BlockSpec((1000, 128), lambda i: (i, 0)),
    )(a, b, c, d)


def kernel(x_users, x_artists, edge_index_a2u, edge_index_u2a):
    nu, na = x_users.shape[0], x_artists.shape[0]
    xu, xa = x_users, x_artists
    us, as_ = [x_users], [x_artists]
    for _ in range(3):
        xu = _scatter_mean(xa, edge_index_a2u[0], edge_index_a2u[1], nu)
        xa = _scatter_mean(xu, edge_index_u2a[0], edge_index_u2a[1], na)
        us.append(xu)
        as_.append(xa)
    return (_combine(*us), _combine(*as_))


# speedup vs baseline: 5.3760x; 5.3760x over previous
import functools

import jax
import jax.numpy as jnp
from jax import lax
from jax.experimental import pallas as pl
from jax.experimental.pallas import tpu as pltpu
from jax.experimental.pallas import tpu_sc as plsc

NROWS = 50000
D = 128
E = 600000
NC, NS, L = 2, 16, 16

DB = 6400
NBLK = 8
NBPC = 4
TRASH = 128
ACC_ROWS = DB + TRASH

MACRO = 2048
BATCH = 128
EPT = 19 * MACRO
EP = NS * EPT

CW = 1024


def _smean_body(table, srcp, dstp, out,
                srcbuf, dstbuf, tmps, tmpd, sidx_send, dloc_send,
                rowsbuf, cnt, cslice, acc, cntstage, sem):
    c = lax.axis_index("c")
    s = lax.axis_index("s")
    zero16 = jnp.zeros((L,), jnp.float32)
    ones16 = jnp.ones((L,), jnp.float32)
    iota16 = lax.iota(jnp.int32, L)

    for b in range(NBPC):
        block = NBPC * c + b
        base = block * DB

        for r in range(64):
            for j in range(D // L):
                rowsbuf[r, pl.ds(j * L, L)] = zero16

        def zero_cnt(i, _):
            cnt[pl.ds(i * L, L)] = zero16
            return 0
        lax.fori_loop(0, DB // L, zero_cnt, 0)

        def zero_tmps(i, _):
            tmps[pl.ds(i * L, L)] = jnp.zeros((L,), jnp.int32)
            return 0
        lax.fori_loop(0, (MACRO + BATCH) // L, zero_tmps, 0)

        zrows = ACC_ROWS // NS
        nz = (zrows + 63) // 64
        for k in range(nz):
            r0 = s * zrows + k * 64
            r0 = pl.multiple_of(jnp.minimum(r0, ACC_ROWS - 64), 8)
            pltpu.sync_copy(rowsbuf.at[pl.ds(0, 64)], acc.at[pl.ds(r0, 64)])
        plsc.subcore_barrier()

        def flush(kofs, fix_p):
            for j in range(BATCH // L):
                sv = tmps[pl.ds(kofs + j * L, L)]
                dv = tmpd[pl.ds(kofs + j * L, L)]
                if fix_p is not None:
                    lane = j * L + iota16
                    keep = lane < fix_p
                    dv = jnp.where(keep, dv, DB + lane)
                sidx_send[pl.ds(j * L, L)] = sv
                dloc_send[pl.ds(j * L, L)] = dv
            pltpu.async_copy(table.at[sidx_send], rowsbuf, sem).wait()
            pltpu.sync_copy(rowsbuf, acc.at[dloc_send], add=True)

        def macro_step(mi, p):
            mbase = pl.multiple_of(s * EPT + mi * MACRO, MACRO)
            pltpu.sync_copy(srcp.at[pl.ds(mbase, MACRO)], srcbuf)
            pltpu.sync_copy(dstp.at[pl.ds(mbase, MACRO)], dstbuf)

            def compact(j, pp):
                d = dstbuf[pl.ds(j * L, L)]
                sv = srcbuf[pl.ds(j * L, L)]
                t = d - base
                inb = (t >= 0) & (t < DB)
                tc = jnp.where(inb, t, 0)
                plsc.addupdate_scatter(cnt, [tc], ones16, mask=inb)
                plsc.store_compressed(tmpd.at[pl.ds(pp, L)], t, mask=inb)
                plsc.store_compressed(tmps.at[pl.ds(pp, L)], sv, mask=inb)
                return pp + jnp.sum(inb.astype(jnp.int32))

            navail = lax.fori_loop(0, MACRO // L, compact, p)
            nb = navail // BATCH

            def flush_k(k, _):
                flush(k * BATCH, None)
                return 0
            lax.fori_loop(0, nb, flush_k, 0)

            rem = navail - nb * BATCH

            @pl.when(nb > 0)
            def _():
                for t_ in range(BATCH // L):
                    sv = tmps[pl.ds(nb * BATCH + t_ * L, L)]
                    dv = tmpd[pl.ds(nb * BATCH + t_ * L, L)]
                    tmps[pl.ds(t_ * L, L)] = sv
                    tmpd[pl.ds(t_ * L, L)] = dv
            return rem

        p_final = lax.fori_loop(0, EPT // MACRO, macro_step, jnp.int32(0))

        @pl.when(p_final > 0)
        def _():
            flush(0, p_final)

        plsc.subcore_barrier()

        pltpu.sync_copy(cnt, cntstage.at[pl.ds(pl.multiple_of(s * DB, 128), DB)])
        plsc.subcore_barrier()

        rows_real = jnp.where(block == NBLK - 1, NROWS - (NBLK - 1) * DB, DB)
        total16 = rows_real // 16
        n16 = (total16 + NS - 1) // NS
        start16 = s * n16
        mych = jnp.clip(total16 - start16, 0, n16)
        start_row = pl.multiple_of(start16 * 16, 16)
        astart = pl.multiple_of(jnp.clip((start_row // 128) * 128, 0, DB - CW), 128)
        off = start_row - astart
        for r in range(NS):
            pltpu.sync_copy(cntstage.at[pl.ds(pl.multiple_of(r * DB + astart, 128), CW)],
                            cslice.at[pl.ds(r * CW, CW)])

        def merge(j, _):
            tot = cslice[pl.ds(off + j * L, L)]
            for r in range(1, NS):
                tot = tot + cslice[pl.ds(r * CW + off + j * L, L)]
            inv = 1.0 / jnp.maximum(tot, 1.0)
            cnt[pl.ds(j * L, L)] = inv
            return 0
        lax.fori_loop(0, mych, merge, 0)

        def out_chunk(q, _):
            r0 = pl.multiple_of(start_row + q * 16, 16)
            pltpu.sync_copy(acc.at[pl.ds(r0, 16)], rowsbuf.at[pl.ds(0, 16)])

            def scale(k, _2):
                r = k // (D // L)
                j = k % (D // L)
                iv = cnt[pl.ds(q * 16 + r, L)][0]
                ivv = jnp.full((L,), iv, jnp.float32)
                rowsbuf[r, pl.ds(j * L, L)] = rowsbuf[r, pl.ds(j * L, L)] * ivv
                return 0
            lax.fori_loop(0, 16 * (D // L), scale, 0)
            pltpu.sync_copy(rowsbuf.at[pl.ds(0, 16)],
                            out.at[pl.ds(base + r0, 16)])
            return 0
        lax.fori_loop(0, mych, out_chunk, 0)
        plsc.subcore_barrier()


@jax.jit
def _smean(table, srcp, dstp):
    mesh = plsc.VectorSubcoreMesh(core_axis_name="c", subcore_axis_name="s",
                                  num_cores=NC, num_subcores=NS)
    f = pl.kernel(
        _smean_body,
        out_type=jax.ShapeDtypeStruct((NROWS, D), jnp.float32),
        mesh=mesh,
        scratch_types=[
            pltpu.VMEM((MACRO,), jnp.int32),
            pltpu.VMEM((MACRO,), jnp.int32),
            pltpu.VMEM((MACRO + BATCH,), jnp.int32),
            pltpu.VMEM((MACRO + BATCH,), jnp.int32),
            pltpu.VMEM((BATCH,), jnp.int32),
            pltpu.VMEM((BATCH,), jnp.int32),
            pltpu.VMEM((BATCH, D), jnp.float32),
            pltpu.VMEM((DB,), jnp.float32),
            pltpu.VMEM((NS * CW,), jnp.float32),
            pltpu.VMEM_SHARED((ACC_ROWS, D), jnp.float32),
            pltpu.VMEM_SHARED((NS * DB,), jnp.float32),
            pltpu.SemaphoreType.DMA,
        ],
        compiler_params=pltpu.CompilerParams(needs_layout_passes=False),
    )
    return f(table, srcp, dstp)


def _pad_edges(e):
    src = e[0].astype(jnp.int32)
    dst = e[1].astype(jnp.int32)
    pad = EP - E
    srcp = jnp.concatenate([src, jnp.zeros((pad,), jnp.int32)])
    dstp = jnp.concatenate([dst, jnp.full((pad,), -1, jnp.int32)])
    return srcp, dstp


def kernel(x_users, x_artists, edge_index_a2u, edge_index_u2a):
    sa, da = _pad_edges(edge_index_a2u)
    su, du = _pad_edges(edge_index_u2a)
    xu, xa = x_users, x_artists
    fu, fa = x_users, x_artists
    for _ in range(3):
        xu = _smean(xa, sa, da)
        xa = _smean(xu, su, du)
        fu = fu + xu
        fa = fa + xa
    return (0.25 * fu, 0.25 * fa)


# speedup vs baseline: 5.7617x; 1.0717x over previous
import functools

import jax
import jax.numpy as jnp
from jax import lax
from jax.experimental import pallas as pl
from jax.experimental.pallas import tpu as pltpu
from jax.experimental.pallas import tpu_sc as plsc

NROWS = 50000
D = 128
E = 600000
NC, NS, L = 2, 16, 16

DB = 6400
NBLK = 8
NBPC = 4
TRASH = 128
ACC_ROWS = DB + TRASH

MACRO = 2048
BATCH = 128
EPT = 19 * MACRO
EP = NS * EPT

CW = 1024


def _smean_body(table, srcp, dstp, out,
                srcbuf, dstbuf, tmps, tmpd, sidx_send, dloc_send,
                sidx2_send, dloc2_send,
                rowsbuf, rows2buf, cnt, cslice, acc, cntstage,
                sem, sem2, ssem, ssem2):
    c = lax.axis_index("c")
    s = lax.axis_index("s")
    zero16 = jnp.zeros((L,), jnp.float32)
    ones16 = jnp.ones((L,), jnp.float32)
    iota16 = lax.iota(jnp.int32, L)

    for b in range(NBPC):
        block = NBPC * c + b
        base = block * DB

        for r in range(64):
            for j in range(D // L):
                rowsbuf[r, pl.ds(j * L, L)] = zero16

        def zero_cnt(i, _):
            cnt[pl.ds(i * L, L)] = zero16
            return 0
        lax.fori_loop(0, DB // L, zero_cnt, 0)

        def zero_tmps(i, _):
            tmps[pl.ds(i * L, L)] = jnp.zeros((L,), jnp.int32)
            return 0
        lax.fori_loop(0, (MACRO + BATCH) // L, zero_tmps, 0)

        zrows = ACC_ROWS // NS
        nz = (zrows + 63) // 64
        for k in range(nz):
            r0 = s * zrows + k * 64
            r0 = pl.multiple_of(jnp.minimum(r0, ACC_ROWS - 64), 8)
            pltpu.sync_copy(rowsbuf.at[pl.ds(0, 64)], acc.at[pl.ds(r0, 64)])
        plsc.subcore_barrier()

        def stage(kofs, sidx_d, dloc_d, fix_p):
            for j in range(BATCH // L):
                sv = tmps[pl.ds(kofs + j * L, L)]
                dv = tmpd[pl.ds(kofs + j * L, L)]
                if fix_p is not None:
                    lane = j * L + iota16
                    keep = lane < fix_p
                    dv = jnp.where(keep, dv, DB + lane)
                sidx_d[pl.ds(j * L, L)] = sv
                dloc_d[pl.ds(j * L, L)] = dv

        def flush(kofs, fix_p):
            stage(kofs, sidx_send, dloc_send, fix_p)
            pltpu.async_copy(table.at[sidx_send], rowsbuf, sem).wait()
            pltpu.sync_copy(rowsbuf, acc.at[dloc_send], add=True)

        def macro_step(mi, p):
            mbase = pl.multiple_of(s * EPT + mi * MACRO, MACRO)
            pltpu.sync_copy(srcp.at[pl.ds(mbase, MACRO)], srcbuf)
            pltpu.sync_copy(dstp.at[pl.ds(mbase, MACRO)], dstbuf)

            def compact(j, pp):
                d = dstbuf[pl.ds(j * L, L)]
                sv = srcbuf[pl.ds(j * L, L)]
                t = d - base
                inb = (t >= 0) & (t < DB)
                tc = jnp.where(inb, t, 0)
                plsc.addupdate_scatter(cnt, [tc], ones16, mask=inb)
                plsc.store_compressed(tmpd.at[pl.ds(pp, L)], t, mask=inb)
                plsc.store_compressed(tmps.at[pl.ds(pp, L)], sv, mask=inb)
                return pp + jnp.sum(inb.astype(jnp.int32))

            navail = lax.fori_loop(0, MACRO // L, compact, p)
            nb = navail // BATCH

            def flush_2(k, _):
                stage(k * 2 * BATCH, sidx_send, dloc_send, None)
                ga = pltpu.async_copy(table.at[sidx_send], rowsbuf, sem)
                stage(k * 2 * BATCH + BATCH, sidx2_send, dloc2_send, None)
                gb = pltpu.async_copy(table.at[sidx2_send], rows2buf, sem2)
                ga.wait()
                sa = pltpu.async_copy(rowsbuf, acc.at[dloc_send], ssem, add=True)
                gb.wait()
                sb = pltpu.async_copy(rows2buf, acc.at[dloc2_send], ssem2, add=True)
                sa.wait()
                sb.wait()
                return 0
            lax.fori_loop(0, nb // 2, flush_2, 0)

            @pl.when(nb % 2 == 1)
            def _():
                flush((nb - 1) * BATCH, None)

            rem = navail - nb * BATCH

            @pl.when(nb > 0)
            def _():
                for t_ in range(BATCH // L):
                    sv = tmps[pl.ds(nb * BATCH + t_ * L, L)]
                    dv = tmpd[pl.ds(nb * BATCH + t_ * L, L)]
                    tmps[pl.ds(t_ * L, L)] = sv
                    tmpd[pl.ds(t_ * L, L)] = dv
            return rem

        p_final = lax.fori_loop(0, EPT // MACRO, macro_step, jnp.int32(0))

        @pl.when(p_final > 0)
        def _():
            flush(0, p_final)

        plsc.subcore_barrier()

        pltpu.sync_copy(cnt, cntstage.at[pl.ds(pl.multiple_of(s * DB, 128), DB)])
        plsc.subcore_barrier()

        rows_real = jnp.where(block == NBLK - 1, NROWS - (NBLK - 1) * DB, DB)
        total16 = rows_real // 16
        n16 = (total16 + NS - 1) // NS
        start16 = s * n16
        mych = jnp.clip(total16 - start16, 0, n16)
        start_row = pl.multiple_of(start16 * 16, 16)
        astart = pl.multiple_of(jnp.clip((start_row // 128) * 128, 0, DB - CW), 128)
        off = start_row - astart
        for r in range(NS):
            pltpu.sync_copy(cntstage.at[pl.ds(pl.multiple_of(r * DB + astart, 128), CW)],
                            cslice.at[pl.ds(r * CW, CW)])

        def merge(j, _):
            tot = cslice[pl.ds(off + j * L, L)]
            for r in range(1, NS):
                tot = tot + cslice[pl.ds(r * CW + off + j * L, L)]
            inv = 1.0 / jnp.maximum(tot, 1.0)
            cnt[pl.ds(j * L, L)] = inv
            return 0
        lax.fori_loop(0, mych, merge, 0)

        def out_chunk(q, _):
            r0 = pl.multiple_of(start_row + q * 16, 16)
            pltpu.sync_copy(acc.at[pl.ds(r0, 16)], rowsbuf.at[pl.ds(0, 16)])

            def scale(k, _2):
                r = k // (D // L)
                j = k % (D // L)
                iv = cnt[pl.ds(q * 16 + r, L)][0]
                ivv = jnp.full((L,), iv, jnp.float32)
                rowsbuf[r, pl.ds(j * L, L)] = rowsbuf[r, pl.ds(j * L, L)] * ivv
                return 0
            lax.fori_loop(0, 16 * (D // L), scale, 0)
            pltpu.sync_copy(rowsbuf.at[pl.ds(0, 16)],
                            out.at[pl.ds(base + r0, 16)])
            return 0
        lax.fori_loop(0, mych, out_chunk, 0)
        plsc.subcore_barrier()


@jax.jit
def _smean(table, srcp, dstp):
    mesh = plsc.VectorSubcoreMesh(core_axis_name="c", subcore_axis_name="s",
                                  num_cores=NC, num_subcores=NS)
    f = pl.kernel(
        _smean_body,
        out_type=jax.ShapeDtypeStruct((NROWS, D), jnp.float32),
        mesh=mesh,
        scratch_types=[
            pltpu.VMEM((MACRO,), jnp.int32),
            pltpu.VMEM((MACRO,), jnp.int32),
            pltpu.VMEM((MACRO + BATCH,), jnp.int32),
            pltpu.VMEM((MACRO + BATCH,), jnp.int32),
            pltpu.VMEM((BATCH,), jnp.int32),
            pltpu.VMEM((BATCH,), jnp.int32),
            pltpu.VMEM((BATCH,), jnp.int32),
            pltpu.VMEM((BATCH,), jnp.int32),
            pltpu.VMEM((BATCH, D), jnp.float32),
            pltpu.VMEM((BATCH, D), jnp.float32),
            pltpu.VMEM((DB,), jnp.float32),
            pltpu.VMEM((NS * CW,), jnp.float32),
            pltpu.VMEM_SHARED((ACC_ROWS, D), jnp.float32),
            pltpu.VMEM_SHARED((NS * DB,), jnp.float32),
            pltpu.SemaphoreType.DMA,
            pltpu.SemaphoreType.DMA,
            pltpu.SemaphoreType.DMA,
            pltpu.SemaphoreType.DMA,
        ],
        compiler_params=pltpu.CompilerParams(needs_layout_passes=False),
    )
    return f(table, srcp, dstp)


def _pad_edges(e):
    src = e[0].astype(jnp.int32)
    dst = e[1].astype(jnp.int32)
    pad = EP - E
    srcp = jnp.concatenate([src, jnp.zeros((pad,), jnp.int32)])
    dstp = jnp.concatenate([dst, jnp.full((pad,), -1, jnp.int32)])
    return srcp, dstp


def kernel(x_users, x_artists, edge_index_a2u, edge_index_u2a):
    sa, da = _pad_edges(edge_index_a2u)
    su, du = _pad_edges(edge_index_u2a)
    xu, xa = x_users, x_artists
    fu, fa = x_users, x_artists
    for _ in range(3):
        xu = _smean(xa, sa, da)
        xa = _smean(xu, su, du)
        fu = fu + xu
        fa = fa + xa
    return (0.25 * fu, 0.25 * fa)


# speedup vs baseline: 6.2706x; 1.0883x over previous
import functools

import jax
import jax.numpy as jnp
from jax import lax
from jax.experimental import pallas as pl
from jax.experimental.pallas import tpu as pltpu
from jax.experimental.pallas import tpu_sc as plsc

NROWS = 50000
D = 128
E = 600000
NC, NS, L = 2, 16, 16

DB = 6400
NBLK = 8
NBPC = 4
TRASH = 128
ACC_ROWS = DB + TRASH

MACRO = 2048
BATCH = 128
EPT = 19 * MACRO
EP = NS * EPT

CW = 1024


def _smean_body(table, srcp, dstp, out,
                srcbuf, dstbuf, tmps, tmpd, sidx_send, dloc_send,
                sidx2_send, dloc2_send,
                rowsbuf, rows2buf, cnt, cslice, acc, cntstage,
                sem, sem2, ssem, ssem2):
    c = lax.axis_index("c")
    s = lax.axis_index("s")
    zero16 = jnp.zeros((L,), jnp.float32)
    ones16 = jnp.ones((L,), jnp.float32)
    iota16 = lax.iota(jnp.int32, L)

    for b in range(NBPC):
        block = NBPC * c + b
        base = block * DB

        for r in range(64):
            for j in range(D // L):
                rowsbuf[r, pl.ds(j * L, L)] = zero16

        def zero_cnt(i, _):
            cnt[pl.ds(i * L, L)] = zero16
            return 0
        lax.fori_loop(0, DB // L, zero_cnt, 0)

        def zero_tmps(i, _):
            tmps[pl.ds(i * L, L)] = jnp.zeros((L,), jnp.int32)
            return 0
        lax.fori_loop(0, (MACRO + BATCH) // L, zero_tmps, 0)

        zrows = ACC_ROWS // NS
        nz = (zrows + 63) // 64
        for k in range(nz):
            r0 = s * zrows + k * 64
            r0 = pl.multiple_of(jnp.minimum(r0, ACC_ROWS - 64), 8)
            pltpu.sync_copy(rowsbuf.at[pl.ds(0, 64)], acc.at[pl.ds(r0, 64)])
        plsc.subcore_barrier()

        def stage(kofs, sidx_d, dloc_d, fix_p):
            for j in range(BATCH // L):
                sv = tmps[pl.ds(kofs + j * L, L)]
                dv = tmpd[pl.ds(kofs + j * L, L)]
                if fix_p is not None:
                    lane = j * L + iota16
                    keep = lane < fix_p
                    dv = jnp.where(keep, dv, DB + lane)
                sidx_d[pl.ds(j * L, L)] = sv
                dloc_d[pl.ds(j * L, L)] = dv

        def flush(kofs, fix_p):
            stage(kofs, sidx_send, dloc_send, fix_p)
            pltpu.async_copy(table.at[sidx_send], rowsbuf, sem).wait()
            pltpu.sync_copy(rowsbuf, acc.at[dloc_send], add=True)

        def macro_step(mi, p):
            mbase = pl.multiple_of(s * EPT + mi * MACRO, MACRO)
            pltpu.sync_copy(srcp.at[pl.ds(mbase, MACRO)], srcbuf)
            pltpu.sync_copy(dstp.at[pl.ds(mbase, MACRO)], dstbuf)

            def compact(j, pp):
                d = dstbuf[pl.ds(j * L, L)]
                sv = srcbuf[pl.ds(j * L, L)]
                t = d - base
                inb = (t >= 0) & (t < DB)
                tc = jnp.where(inb, t, 0)
                plsc.addupdate_scatter(cnt, [tc], ones16, mask=inb)
                plsc.store_compressed(tmpd.at[pl.ds(pp, L)], t, mask=inb)
                plsc.store_compressed(tmps.at[pl.ds(pp, L)], sv, mask=inb)
                return pp + jnp.sum(inb.astype(jnp.int32))

            navail = lax.fori_loop(0, MACRO // L, compact, p)
            nb = navail // BATCH

            def flush_2(k, _):
                stage(k * 2 * BATCH, sidx_send, dloc_send, None)
                ga = pltpu.async_copy(table.at[sidx_send], rowsbuf, sem)
                stage(k * 2 * BATCH + BATCH, sidx2_send, dloc2_send, None)
                gb = pltpu.async_copy(table.at[sidx2_send], rows2buf, sem2)
                ga.wait()
                sa = pltpu.async_copy(rowsbuf, acc.at[dloc_send], ssem, add=True)
                gb.wait()
                sb = pltpu.async_copy(rows2buf, acc.at[dloc2_send], ssem2, add=True)
                sa.wait()
                sb.wait()
                return 0
            lax.fori_loop(0, nb // 2, flush_2, 0)

            @pl.when(nb % 2 == 1)
            def _():
                flush((nb - 1) * BATCH, None)

            rem = navail - nb * BATCH

            @pl.when(nb > 0)
            def _():
                for t_ in range(BATCH // L):
                    sv = tmps[pl.ds(nb * BATCH + t_ * L, L)]
                    dv = tmpd[pl.ds(nb * BATCH + t_ * L, L)]
                    tmps[pl.ds(t_ * L, L)] = sv
                    tmpd[pl.ds(t_ * L, L)] = dv
            return rem

        p_final = lax.fori_loop(0, EPT // MACRO, macro_step, jnp.int32(0))

        @pl.when(p_final > 0)
        def _():
            flush(0, p_final)

        plsc.subcore_barrier()

        pltpu.sync_copy(cnt, cntstage.at[pl.ds(pl.multiple_of(s * DB, 128), DB)])
        plsc.subcore_barrier()

        rows_real = jnp.where(block == NBLK - 1, NROWS - (NBLK - 1) * DB, DB)
        total16 = rows_real // 16
        n16 = (total16 + NS - 1) // NS
        start16 = s * n16
        mych = jnp.clip(total16 - start16, 0, n16)
        start_row = pl.multiple_of(start16 * 16, 16)
        astart = pl.multiple_of(jnp.clip((start_row // 128) * 128, 0, DB - CW), 128)
        off = start_row - astart
        for r in range(NS):
            pltpu.sync_copy(cntstage.at[pl.ds(pl.multiple_of(r * DB + astart, 128), CW)],
                            cslice.at[pl.ds(r * CW, CW)])

        def merge(j, _):
            tot = cslice[pl.ds(off + j * L, L)]
            for r in range(1, NS):
                tot = tot + cslice[pl.ds(r * CW + off + j * L, L)]
            inv = 1.0 / jnp.maximum(tot, 1.0)
            cnt[pl.ds(j * L, L)] = inv
            return 0
        lax.fori_loop(0, mych, merge, 0)

        def out_chunk(q, _):
            r0 = pl.multiple_of(start_row + q * 16, 16)
            pltpu.sync_copy(acc.at[pl.ds(r0, 16)], rowsbuf.at[pl.ds(0, 16)])

            def scale(r, _2):
                iv = cnt[pl.ds(q * 16 + r, L)][0]
                ivv = jnp.full((L,), iv, jnp.float32)
                for j in range(D // L):
                    rowsbuf[r, pl.ds(j * L, L)] = rowsbuf[r, pl.ds(j * L, L)] * ivv
                return 0
            lax.fori_loop(0, 16, scale, 0)
            pltpu.sync_copy(rowsbuf.at[pl.ds(0, 16)],
                            out.at[pl.ds(base + r0, 16)])
            return 0
        lax.fori_loop(0, mych, out_chunk, 0)
        plsc.subcore_barrier()


@jax.jit
def _smean(table, srcp, dstp):
    mesh = plsc.VectorSubcoreMesh(core_axis_name="c", subcore_axis_name="s",
                                  num_cores=NC, num_subcores=NS)
    f = pl.kernel(
        _smean_body,
        out_type=jax.ShapeDtypeStruct((NROWS, D), jnp.float32),
        mesh=mesh,
        scratch_types=[
            pltpu.VMEM((MACRO,), jnp.int32),
            pltpu.VMEM((MACRO,), jnp.int32),
            pltpu.VMEM((MACRO + BATCH,), jnp.int32),
            pltpu.VMEM((MACRO + BATCH,), jnp.int32),
            pltpu.VMEM((BATCH,), jnp.int32),
            pltpu.VMEM((BATCH,), jnp.int32),
            pltpu.VMEM((BATCH,), jnp.int32),
            pltpu.VMEM((BATCH,), jnp.int32),
            pltpu.VMEM((BATCH, D), jnp.float32),
            pltpu.VMEM((BATCH, D), jnp.float32),
            pltpu.VMEM((DB,), jnp.float32),
            pltpu.VMEM((NS * CW,), jnp.float32),
            pltpu.VMEM_SHARED((ACC_ROWS, D), jnp.float32),
            pltpu.VMEM_SHARED((NS * DB,), jnp.float32),
            pltpu.SemaphoreType.DMA,
            pltpu.SemaphoreType.DMA,
            pltpu.SemaphoreType.DMA,
            pltpu.SemaphoreType.DMA,
        ],
        compiler_params=pltpu.CompilerParams(needs_layout_passes=False),
    )
    return f(table, srcp, dstp)


def _pad_edges(e):
    src = e[0].astype(jnp.int32)
    dst = e[1].astype(jnp.int32)
    pad = EP - E
    srcp = jnp.concatenate([src, jnp.zeros((pad,), jnp.int32)])
    dstp = jnp.concatenate([dst, jnp.full((pad,), -1, jnp.int32)])
    return srcp, dstp


def kernel(x_users, x_artists, edge_index_a2u, edge_index_u2a):
    sa, da = _pad_edges(edge_index_a2u)
    su, du = _pad_edges(edge_index_u2a)
    xu, xa = x_users, x_artists
    fu, fa = x_users, x_artists
    for _ in range(3):
        xu = _smean(xa, sa, da)
        xa = _smean(xu, su, du)
        fu = fu + xu
        fa = fa + xa
    return (0.25 * fu, 0.25 * fa)


# speedup vs baseline: 7.5228x; 1.1997x over previous
import jax
import jax.numpy as jnp
from jax import lax
from jax.experimental import pallas as pl
from jax.experimental.pallas import tpu as pltpu
from jax.experimental.pallas import tpu_sc as plsc

NROWS = 50000
D = 128
E = 600000
NC, NS, L = 2, 16, 16

DB = 6400
NBLK = 8
NBPC = 4
TRASH = 128
ACC_ROWS = DB + TRASH

MACRO = 2048
BATCH = 128
EPT = 19 * MACRO
EP = NS * EPT
SEGCAP = EPT + MACRO

CW = 1024


def _zero_rows64(rowsbuf, zero16):
    for r in range(64):
        for j in range(D // L):
            rowsbuf[r, pl.ds(j * L, L)] = zero16


def _zero_acc(s, acc, rowsbuf):
    zrows = ACC_ROWS // NS
    nz = (zrows + 63) // 64
    for k in range(nz):
        r0 = s * zrows + k * 64
        r0 = pl.multiple_of(jnp.minimum(r0, ACC_ROWS - 64), 8)
        pltpu.sync_copy(rowsbuf.at[pl.ds(0, 64)], acc.at[pl.ds(r0, 64)])


def _own_rows(s, block):
    rows_real = jnp.where(block == NBLK - 1, NROWS - (NBLK - 1) * DB, DB)
    total16 = rows_real // 16
    n16 = (total16 + NS - 1) // NS
    start16 = s * n16
    mych = jnp.clip(total16 - start16, 0, n16)
    start_row = pl.multiple_of(start16 * 16, 16)
    return mych, start_row


def _scale_rows(q, cnt, rowsbuf):
    def scale(r, _2):
        iv = cnt[pl.ds(q * 16 + r, L)][0]
        ivv = jnp.full((L,), iv, jnp.float32)
        for j in range(D // L):
            rowsbuf[r, pl.ds(j * L, L)] = rowsbuf[r, pl.ds(j * L, L)] * ivv
        return 0
    lax.fori_loop(0, 16, scale, 0)


def _scan_body(table, srcp, dstp, out, srcc, dlocc, counts, invout,
               srcbuf, dstbuf, tmps, tmpd, sidx_send, dloc_send,
               sidx2_send, dloc2_send, rowsbuf, rows2buf, cnt, cslice, cntw,
               acc, cntstage, sem, sem2, ssem, ssem2, wsem, wsem2):
    c = lax.axis_index("c")
    s = lax.axis_index("s")
    zero16 = jnp.zeros((L,), jnp.float32)
    ones16 = jnp.ones((L,), jnp.float32)
    iota16 = lax.iota(jnp.int32, L)

    for b in range(NBPC):
        block = NBPC * c + b
        base = block * DB
        seg = pl.multiple_of((block * NS + s) * SEGCAP, 128)

        _zero_rows64(rowsbuf, zero16)

        def zero_cnt(i, _):
            cnt[pl.ds(i * L, L)] = zero16
            return 0
        lax.fori_loop(0, DB // L, zero_cnt, 0)

        def zero_tmps(i, _):
            tmps[pl.ds(i * L, L)] = jnp.zeros((L,), jnp.int32)
            return 0
        lax.fori_loop(0, (MACRO + BATCH) // L, zero_tmps, 0)

        _zero_acc(s, acc, rowsbuf)
        plsc.subcore_barrier()

        def stage(kofs, sidx_d, dloc_d, fix_p):
            for j in range(BATCH // L):
                sv = tmps[pl.ds(kofs + j * L, L)]
                dv = tmpd[pl.ds(kofs + j * L, L)]
                if fix_p is not None:
                    lane = j * L + iota16
                    keep = lane < fix_p
                    dv = jnp.where(keep, dv, DB + lane)
                sidx_d[pl.ds(j * L, L)] = sv
                dloc_d[pl.ds(j * L, L)] = dv

        def flush(kofs, fix_p):
            stage(kofs, sidx_send, dloc_send, fix_p)
            pltpu.async_copy(table.at[sidx_send], rowsbuf, sem).wait()
            pltpu.sync_copy(rowsbuf, acc.at[dloc_send], add=True)

        def macro_step(mi, carry):
            p, wofs = carry
            mbase = pl.multiple_of(s * EPT + mi * MACRO, MACRO)
            pltpu.sync_copy(srcp.at[pl.ds(mbase, MACRO)], srcbuf)
            pltpu.sync_copy(dstp.at[pl.ds(mbase, MACRO)], dstbuf)

            def compact(j, pp):
                d = dstbuf[pl.ds(j * L, L)]
                sv = srcbuf[pl.ds(j * L, L)]
                t = d - base
                inb = (t >= 0) & (t < DB)
                tc = jnp.where(inb, t, 0)
                plsc.addupdate_scatter(cnt, [tc], ones16, mask=inb)
                plsc.store_compressed(tmpd.at[pl.ds(pp, L)], t, mask=inb)
                plsc.store_compressed(tmps.at[pl.ds(pp, L)], sv, mask=inb)
                return pp + jnp.sum(inb.astype(jnp.int32))

            navail = lax.fori_loop(0, MACRO // L, compact, p)
            nb = navail // BATCH

            aofs = pl.multiple_of(seg + wofs * BATCH, 8)
            w1 = pltpu.async_copy(tmps.at[pl.ds(0, MACRO)],
                                  srcc.at[pl.ds(aofs, MACRO)], wsem)
            w2 = pltpu.async_copy(tmpd.at[pl.ds(0, MACRO)],
                                  dlocc.at[pl.ds(aofs, MACRO)], wsem2)

            def flush_2(k, _):
                stage(k * 2 * BATCH, sidx_send, dloc_send, None)
                ga = pltpu.async_copy(table.at[sidx_send], rowsbuf, sem)
                stage(k * 2 * BATCH + BATCH, sidx2_send, dloc2_send, None)
                gb = pltpu.async_copy(table.at[sidx2_send], rows2buf, sem2)
                ga.wait()
                sa = pltpu.async_copy(rowsbuf, acc.at[dloc_send], ssem, add=True)
                gb.wait()
                sb = pltpu.async_copy(rows2buf, acc.at[dloc2_send], ssem2, add=True)
                sa.wait()
                sb.wait()
                return 0
            lax.fori_loop(0, nb // 2, flush_2, 0)

            @pl.when(nb % 2 == 1)
            def _():
                flush((nb - 1) * BATCH, None)

            w1.wait()
            w2.wait()

            rem = navail - nb * BATCH

            @pl.when(nb > 0)
            def _():
                for t_ in range(BATCH // L):
                    sv = tmps[pl.ds(nb * BATCH + t_ * L, L)]
                    dv = tmpd[pl.ds(nb * BATCH + t_ * L, L)]
                    tmps[pl.ds(t_ * L, L)] = sv
                    tmpd[pl.ds(t_ * L, L)] = dv
            return rem, wofs + nb

        p_final, wofs_final = lax.fori_loop(
            0, EPT // MACRO, macro_step, (jnp.int32(0), jnp.int32(0)))

        @pl.when(p_final > 0)
        def _():
            flush(0, p_final)

        n_tb = wofs_final * BATCH + p_final
        cv = cntw[pl.ds(0, L)]
        cntw[pl.ds(0, L)] = jnp.where(iota16 == b, n_tb, cv)

        plsc.subcore_barrier()

        pltpu.sync_copy(cnt, cntstage.at[pl.ds(pl.multiple_of(s * DB, 128), DB)])
        plsc.subcore_barrier()

        mych, start_row = _own_rows(s, block)
        astart = pl.multiple_of(jnp.clip((start_row // 128) * 128, 0, DB - CW), 128)
        off = start_row - astart
        for r in range(NS):
            pltpu.sync_copy(cntstage.at[pl.ds(pl.multiple_of(r * DB + astart, 128), CW)],
                            cslice.at[pl.ds(r * CW, CW)])

        def merge(j, _):
            tot = cslice[pl.ds(off + j * L, L)]
            for r in range(1, NS):
                tot = tot + cslice[pl.ds(r * CW + off + j * L, L)]
            inv = 1.0 / jnp.maximum(tot, 1.0)
            cnt[pl.ds(j * L, L)] = inv
            return 0
        lax.fori_loop(0, mych, merge, 0)

        def out_chunk(q, _):
            r0 = pl.multiple_of(start_row + q * 16, 16)
            pltpu.sync_copy(acc.at[pl.ds(r0, 16)], rowsbuf.at[pl.ds(0, 16)])
            _scale_rows(q, cnt, rowsbuf)
            pltpu.sync_copy(rowsbuf.at[pl.ds(0, 16)],
                            out.at[pl.ds(base + r0, 16)])
            pltpu.sync_copy(cnt.at[pl.ds(pl.multiple_of(q * 16, 16), 16)],
                            invout.at[pl.ds(base + r0, 16)])
            return 0
        lax.fori_loop(0, mych, out_chunk, 0)
        plsc.subcore_barrier()

    pltpu.sync_copy(
        cntw, counts.at[pl.ds(pl.multiple_of((c * NS + s) * L, 16), L)])


def _replay_body(table, srcc, dlocc, counts, invin, out,
                 srcbuf, dstbuf, sidx_send, dloc_send,
                 sidx2_send, dloc2_send, rowsbuf, rows2buf, cnt, cntw,
                 acc, sem, sem2, ssem, ssem2):
    c = lax.axis_index("c")
    s = lax.axis_index("s")
    zero16 = jnp.zeros((L,), jnp.float32)
    iota16 = lax.iota(jnp.int32, L)

    pltpu.sync_copy(
        counts.at[pl.ds(pl.multiple_of((c * NS + s) * L, 16), L)], cntw)

    for b in range(NBPC):
        block = NBPC * c + b
        base = block * DB
        seg = pl.multiple_of((block * NS + s) * SEGCAP, 128)

        _zero_rows64(rowsbuf, zero16)
        _zero_acc(s, acc, rowsbuf)
        plsc.subcore_barrier()

        n_tb = cntw[pl.ds(0, L)][b]
        nbat = (n_tb + BATCH - 1) // BATCH

        def stage_r(lofs, gofs, sidx_d, dloc_d):
            fp = n_tb - gofs * BATCH
            for j in range(BATCH // L):
                sv = srcbuf[pl.ds(lofs * BATCH + j * L, L)]
                dv = dstbuf[pl.ds(lofs * BATCH + j * L, L)]
                lane = j * L + iota16
                dv = jnp.where(lane < fp, dv, DB + lane)
                sidx_d[pl.ds(j * L, L)] = sv
                dloc_d[pl.ds(j * L, L)] = dv

        def chunk_step(mi, _):
            cofs = pl.multiple_of(seg + mi * MACRO, 8)
            pltpu.sync_copy(srcc.at[pl.ds(cofs, MACRO)], srcbuf)
            pltpu.sync_copy(dlocc.at[pl.ds(cofs, MACRO)], dstbuf)
            nb_c = jnp.minimum(nbat - mi * (MACRO // BATCH), MACRO // BATCH)

            def flush_2(k, _2):
                g0 = mi * (MACRO // BATCH) + 2 * k
                stage_r(2 * k, g0, sidx_send, dloc_send)
                ga = pltpu.async_copy(table.at[sidx_send], rowsbuf, sem)
                stage_r(2 * k + 1, g0 + 1, sidx2_send, dloc2_send)
                gb = pltpu.async_copy(table.at[sidx2_send], rows2buf, sem2)
                ga.wait()
                sa = pltpu.async_copy(rowsbuf, acc.at[dloc_send], ssem, add=True)
                gb.wait()
                sb = pltpu.async_copy(rows2buf, acc.at[dloc2_send], ssem2, add=True)
                sa.wait()
                sb.wait()
                return 0
            lax.fori_loop(0, nb_c // 2, flush_2, 0)

            @pl.when(nb_c % 2 == 1)
            def _():
                k = nb_c - 1
                stage_r(k, mi * (MACRO // BATCH) + k, sidx_send, dloc_send)
                pltpu.async_copy(table.at[sidx_send], rowsbuf, sem).wait()
                pltpu.sync_copy(rowsbuf, acc.at[dloc_send], add=True)
            return 0
        nch = (nbat + (MACRO // BATCH) - 1) // (MACRO // BATCH)
        lax.fori_loop(0, nch, chunk_step, 0)
        plsc.subcore_barrier()

        mych, start_row = _own_rows(s, block)

        def out_chunk(q, _):
            r0 = pl.multiple_of(start_row + q * 16, 16)
            pltpu.sync_copy(acc.at[pl.ds(r0, 16)], rowsbuf.at[pl.ds(0, 16)])
            pltpu.sync_copy(invin.at[pl.ds(base + r0, 16)],
                            cnt.at[pl.ds(pl.multiple_of(q * 16, 16), 16)])
            _scale_rows(q, cnt, rowsbuf)
            pltpu.sync_copy(rowsbuf.at[pl.ds(0, 16)],
                            out.at[pl.ds(base + r0, 16)])
            return 0
        lax.fori_loop(0, mych, out_chunk, 0)
        plsc.subcore_barrier()


_MESH = plsc.VectorSubcoreMesh(core_axis_name="c", subcore_axis_name="s",
                               num_cores=NC, num_subcores=NS)
_SEGTOT = NBLK * NS * SEGCAP


@jax.jit
def _smean_scan(table, srcp, dstp):
    f = pl.kernel(
        _scan_body,
        out_type=(
            jax.ShapeDtypeStruct((NROWS, D), jnp.float32),
            jax.ShapeDtypeStruct((_SEGTOT,), jnp.int32),
            jax.ShapeDtypeStruct((_SEGTOT,), jnp.int32),
            jax.ShapeDtypeStruct((NC * NS * L,), jnp.int32),
            jax.ShapeDtypeStruct((NROWS,), jnp.float32),
        ),
        mesh=_MESH,
        scratch_types=[
            pltpu.VMEM((MACRO,), jnp.int32),
            pltpu.VMEM((MACRO,), jnp.int32),
            pltpu.VMEM((MACRO + BATCH,), jnp.int32),
            pltpu.VMEM((MACRO + BATCH,), jnp.int32),
            pltpu.VMEM((BATCH,), jnp.int32),
            pltpu.VMEM((BATCH,), jnp.int32),
            pltpu.VMEM((BATCH,), jnp.int32),
            pltpu.VMEM((BATCH,), jnp.int32),
            pltpu.VMEM((BATCH, D), jnp.float32),
            pltpu.VMEM((BATCH, D), jnp.float32),
            pltpu.VMEM((DB,), jnp.float32),
            pltpu.VMEM((NS * CW,), jnp.float32),
            pltpu.VMEM((L,), jnp.int32),
            pltpu.VMEM_SHARED((ACC_ROWS, D), jnp.float32),
            pltpu.VMEM_SHARED((NS * DB,), jnp.float32),
            pltpu.SemaphoreType.DMA,
            pltpu.SemaphoreType.DMA,
            pltpu.SemaphoreType.DMA,
            pltpu.SemaphoreType.DMA,
            pltpu.SemaphoreType.DMA,
            pltpu.SemaphoreType.DMA,
        ],
        compiler_params=pltpu.CompilerParams(needs_layout_passes=False),
    )
    return f(table, srcp, dstp)


@jax.jit
def _smean_replay(table, srcc, dlocc, counts, invin):
    f = pl.kernel(
        _replay_body,
        out_type=jax.ShapeDtypeStruct((NROWS, D), jnp.float32),
        mesh=_MESH,
        scratch_types=[
            pltpu.VMEM((MACRO,), jnp.int32),
            pltpu.VMEM((MACRO,), jnp.int32),
            pltpu.VMEM((BATCH,), jnp.int32),
            pltpu.VMEM((BATCH,), jnp.int32),
            pltpu.VMEM((BATCH,), jnp.int32),
            pltpu.VMEM((BATCH,), jnp.int32),
            pltpu.VMEM((BATCH, D), jnp.float32),
            pltpu.VMEM((BATCH, D), jnp.float32),
            pltpu.VMEM((DB,), jnp.float32),
            pltpu.VMEM((L,), jnp.int32),
            pltpu.VMEM_SHARED((ACC_ROWS, D), jnp.float32),
            pltpu.SemaphoreType.DMA,
            pltpu.SemaphoreType.DMA,
            pltpu.SemaphoreType.DMA,
            pltpu.SemaphoreType.DMA,
        ],
        compiler_params=pltpu.CompilerParams(needs_layout_passes=False),
    )
    return f(table, srcc, dlocc, counts, invin)


def _pad_edges(e):
    src = e[0].astype(jnp.int32)
    dst = e[1].astype(jnp.int32)
    pad = EP - E
    srcp = jnp.concatenate([src, jnp.zeros((pad,), jnp.int32)])
    dstp = jnp.concatenate([dst, jnp.full((pad,), -1, jnp.int32)])
    return srcp, dstp


def kernel(x_users, x_artists, edge_index_a2u, edge_index_u2a):
    sa, da = _pad_edges(edge_index_a2u)
    su, du = _pad_edges(edge_index_u2a)
    xu, xa = x_users, x_artists
    xu, a_srcc, a_dlocc, a_counts, a_inv = _smean_scan(xa, sa, da)
    xa, u_srcc, u_dlocc, u_counts, u_inv = _smean_scan(xu, su, du)
    fu = x_users + xu
    fa = x_artists + xa
    for _ in range(2):
        xu = _smean_replay(xa, a_srcc, a_dlocc, a_counts, a_inv)
        xa = _smean_replay(xu, u_srcc, u_dlocc, u_counts, u_inv)
        fu = fu + xu
        fa = fa + xa
    return (0.25 * fu, 0.25 * fa)


# speedup vs baseline: 7.7702x; 1.0329x over previous
import jax
import jax.numpy as jnp
from jax import lax
from jax.experimental import pallas as pl
from jax.experimental.pallas import tpu as pltpu
from jax.experimental.pallas import tpu_sc as plsc

NROWS = 50000
D = 128
E = 600000
NC, NS, L = 2, 16, 16

DB = 6400
NBLK = 8
NBPC = 4
TRASH = 128
ACC_ROWS = DB + TRASH

MACRO = 2048
BATCH = 128
EPT = 19 * MACRO
EP = NS * EPT
SEGCAP = EPT + MACRO

CW = 1024


def _zero_rows64(rowsbuf, zero16):
    for r in range(64):
        for j in range(D // L):
            rowsbuf[r, pl.ds(j * L, L)] = zero16


def _zero_acc(s, acc, rowsbuf):
    zrows = ACC_ROWS // NS
    nz = (zrows + 63) // 64
    for k in range(nz):
        r0 = s * zrows + k * 64
        r0 = pl.multiple_of(jnp.minimum(r0, ACC_ROWS - 64), 8)
        pltpu.sync_copy(rowsbuf.at[pl.ds(0, 64)], acc.at[pl.ds(r0, 64)])


def _own_rows(s, block):
    rows_real = jnp.where(block == NBLK - 1, NROWS - (NBLK - 1) * DB, DB)
    total16 = rows_real // 16
    n16 = (total16 + NS - 1) // NS
    start16 = s * n16
    mych = jnp.clip(total16 - start16, 0, n16)
    start_row = pl.multiple_of(start16 * 16, 16)
    return mych, start_row


def _scale_rows(q, cnt, rowsbuf):
    def scale(r, _2):
        iv = cnt[pl.ds(q * 16 + r, L)][0]
        ivv = jnp.full((L,), iv, jnp.float32)
        for j in range(D // L):
            rowsbuf[r, pl.ds(j * L, L)] = rowsbuf[r, pl.ds(j * L, L)] * ivv
        return 0
    lax.fori_loop(0, 16, scale, 0)


def _scan_body(table, srcp, dstp, out, srcc, dlocc, counts, invout,
               srcbuf, dstbuf, tmps, tmpd, sidx_send, dloc_send,
               sidx2_send, dloc2_send, sidx3_send, dloc3_send,
               sidx4_send, dloc4_send, rowsbuf, rows2buf, rows3buf, rows4buf,
               cnt, cslice, cntw, acc, cntstage,
               sem, sem2, sem3, sem4, ssem, ssem2, ssem3, ssem4, wsem, wsem2):
    c = lax.axis_index("c")
    s = lax.axis_index("s")
    slots = [(sidx_send, dloc_send, rowsbuf, sem, ssem),
             (sidx2_send, dloc2_send, rows2buf, sem2, ssem2),
             (sidx3_send, dloc3_send, rows3buf, sem3, ssem3),
             (sidx4_send, dloc4_send, rows4buf, sem4, ssem4)]
    zero16 = jnp.zeros((L,), jnp.float32)
    ones16 = jnp.ones((L,), jnp.float32)
    iota16 = lax.iota(jnp.int32, L)

    for b in range(NBPC):
        block = NBPC * c + b
        base = block * DB
        seg = pl.multiple_of((block * NS + s) * SEGCAP, 128)

        _zero_rows64(rowsbuf, zero16)

        def zero_cnt(i, _):
            cnt[pl.ds(i * L, L)] = zero16
            return 0
        lax.fori_loop(0, DB // L, zero_cnt, 0)

        def zero_tmps(i, _):
            tmps[pl.ds(i * L, L)] = jnp.zeros((L,), jnp.int32)
            return 0
        lax.fori_loop(0, (MACRO + BATCH) // L, zero_tmps, 0)

        _zero_acc(s, acc, rowsbuf)
        plsc.subcore_barrier()

        def stage(kofs, sidx_d, dloc_d, fix_p):
            for j in range(BATCH // L):
                sv = tmps[pl.ds(kofs + j * L, L)]
                dv = tmpd[pl.ds(kofs + j * L, L)]
                if fix_p is not None:
                    lane = j * L + iota16
                    keep = lane < fix_p
                    dv = jnp.where(keep, dv, DB + lane)
                sidx_d[pl.ds(j * L, L)] = sv
                dloc_d[pl.ds(j * L, L)] = dv

        def flush(kofs, fix_p):
            stage(kofs, sidx_send, dloc_send, fix_p)
            pltpu.async_copy(table.at[sidx_send], rowsbuf, sem).wait()
            pltpu.sync_copy(rowsbuf, acc.at[dloc_send], add=True)

        def macro_step(mi, carry):
            p, wofs = carry
            mbase = pl.multiple_of(s * EPT + mi * MACRO, MACRO)
            pltpu.sync_copy(srcp.at[pl.ds(mbase, MACRO)], srcbuf)
            pltpu.sync_copy(dstp.at[pl.ds(mbase, MACRO)], dstbuf)

            def compact(j, pp):
                d = dstbuf[pl.ds(j * L, L)]
                sv = srcbuf[pl.ds(j * L, L)]
                t = d - base
                inb = (t >= 0) & (t < DB)
                tc = jnp.where(inb, t, 0)
                plsc.addupdate_scatter(cnt, [tc], ones16, mask=inb)
                plsc.store_compressed(tmpd.at[pl.ds(pp, L)], t, mask=inb)
                plsc.store_compressed(tmps.at[pl.ds(pp, L)], sv, mask=inb)
                return pp + jnp.sum(inb.astype(jnp.int32))

            navail = lax.fori_loop(0, MACRO // L, compact, p)
            nb = navail // BATCH

            aofs = pl.multiple_of(seg + wofs * BATCH, 8)
            w1 = pltpu.async_copy(tmps.at[pl.ds(0, MACRO)],
                                  srcc.at[pl.ds(aofs, MACRO)], wsem)
            w2 = pltpu.async_copy(tmpd.at[pl.ds(0, MACRO)],
                                  dlocc.at[pl.ds(aofs, MACRO)], wsem2)

            def flush_n(kbase, nslots):
                gs = []
                for i in range(nslots):
                    sd, dd, rb, gsm, ssm = slots[i]
                    stage(kbase + i * BATCH, sd, dd, None)
                    gs.append(pltpu.async_copy(table.at[sd], rb, gsm))
                ss = []
                for i in range(nslots):
                    sd, dd, rb, gsm, ssm = slots[i]
                    gs[i].wait()
                    ss.append(pltpu.async_copy(rb, acc.at[dd], ssm, add=True))
                for d_ in ss:
                    d_.wait()

            def flush_2(k, _):
                flush_n(k * 2 * BATCH, 2)
                return 0
            lax.fori_loop(0, nb // 2, flush_2, 0)

            @pl.when(nb % 2 == 1)
            def _():
                flush((nb - 1) * BATCH, None)

            w1.wait()
            w2.wait()

            rem = navail - nb * BATCH

            @pl.when(nb > 0)
            def _():
                for t_ in range(BATCH // L):
                    sv = tmps[pl.ds(nb * BATCH + t_ * L, L)]
                    dv = tmpd[pl.ds(nb * BATCH + t_ * L, L)]
                    tmps[pl.ds(t_ * L, L)] = sv
                    tmpd[pl.ds(t_ * L, L)] = dv
            return rem, wofs + nb

        p_final, wofs_final = lax.fori_loop(
            0, EPT // MACRO, macro_step, (jnp.int32(0), jnp.int32(0)))

        @pl.when(p_final > 0)
        def _():
            flush(0, p_final)

        n_tb = wofs_final * BATCH + p_final
        cv = cntw[pl.ds(0, L)]
        cntw[pl.ds(0, L)] = jnp.where(iota16 == b, n_tb, cv)

        plsc.subcore_barrier()

        pltpu.sync_copy(cnt, cntstage.at[pl.ds(pl.multiple_of(s * DB, 128), DB)])
        plsc.subcore_barrier()

        mych, start_row = _own_rows(s, block)
        astart = pl.multiple_of(jnp.clip((start_row // 128) * 128, 0, DB - CW), 128)
        off = start_row - astart
        for r in range(NS):
            pltpu.sync_copy(cntstage.at[pl.ds(pl.multiple_of(r * DB + astart, 128), CW)],
                            cslice.at[pl.ds(r * CW, CW)])

        def merge(j, _):
            tot = cslice[pl.ds(off + j * L, L)]
            for r in range(1, NS):
                tot = tot + cslice[pl.ds(r * CW + off + j * L, L)]
            inv = 1.0 / jnp.maximum(tot, 1.0)
            cnt[pl.ds(j * L, L)] = inv
            return 0
        lax.fori_loop(0, mych, merge, 0)

        def out_chunk(q, _):
            r0 = pl.multiple_of(start_row + q * 16, 16)
            pltpu.sync_copy(acc.at[pl.ds(r0, 16)], rowsbuf.at[pl.ds(0, 16)])
            _scale_rows(q, cnt, rowsbuf)
            pltpu.sync_copy(rowsbuf.at[pl.ds(0, 16)],
                            out.at[pl.ds(base + r0, 16)])
            pltpu.sync_copy(cnt.at[pl.ds(pl.multiple_of(q * 16, 16), 16)],
                            invout.at[pl.ds(base + r0, 16)])
            return 0
        lax.fori_loop(0, mych, out_chunk, 0)
        plsc.subcore_barrier()

    pltpu.sync_copy(
        cntw, counts.at[pl.ds(pl.multiple_of((c * NS + s) * L, 16), L)])


def _replay_body(table, srcc, dlocc, counts, invin, out,
                 srcbuf, dstbuf, sidx_send, dloc_send,
                 sidx2_send, dloc2_send, sidx3_send, dloc3_send,
                 sidx4_send, dloc4_send, rowsbuf, rows2buf, rows3buf, rows4buf,
                 cnt, cntw, acc,
                 sem, sem2, sem3, sem4, ssem, ssem2, ssem3, ssem4):
    c = lax.axis_index("c")
    s = lax.axis_index("s")
    slots = [(sidx_send, dloc_send, rowsbuf, sem, ssem),
             (sidx2_send, dloc2_send, rows2buf, sem2, ssem2),
             (sidx3_send, dloc3_send, rows3buf, sem3, ssem3),
             (sidx4_send, dloc4_send, rows4buf, sem4, ssem4)]
    zero16 = jnp.zeros((L,), jnp.float32)
    iota16 = lax.iota(jnp.int32, L)

    pltpu.sync_copy(
        counts.at[pl.ds(pl.multiple_of((c * NS + s) * L, 16), L)], cntw)

    for b in range(NBPC):
        block = NBPC * c + b
        base = block * DB
        seg = pl.multiple_of((block * NS + s) * SEGCAP, 128)

        _zero_rows64(rowsbuf, zero16)
        _zero_acc(s, acc, rowsbuf)
        plsc.subcore_barrier()

        n_tb = cntw[pl.ds(0, L)][b]
        nbat = (n_tb + BATCH - 1) // BATCH

        def stage_r(lofs, gofs, sidx_d, dloc_d):
            fp = n_tb - gofs * BATCH
            for j in range(BATCH // L):
                sv = srcbuf[pl.ds(lofs * BATCH + j * L, L)]
                dv = dstbuf[pl.ds(lofs * BATCH + j * L, L)]
                lane = j * L + iota16
                dv = jnp.where(lane < fp, dv, DB + lane)
                sidx_d[pl.ds(j * L, L)] = sv
                dloc_d[pl.ds(j * L, L)] = dv

        def chunk_step(mi, _):
            cofs = pl.multiple_of(seg + mi * MACRO, 8)
            pltpu.sync_copy(srcc.at[pl.ds(cofs, MACRO)], srcbuf)
            pltpu.sync_copy(dlocc.at[pl.ds(cofs, MACRO)], dstbuf)
            nb_c = jnp.minimum(nbat - mi * (MACRO // BATCH), MACRO // BATCH)

            def flush_n(lbase, nslots):
                gs = []
                for i in range(nslots):
                    sd, dd, rb, gsm, ssm = slots[i]
                    stage_r(lbase + i, mi * (MACRO // BATCH) + lbase + i, sd, dd)
                    gs.append(pltpu.async_copy(table.at[sd], rb, gsm))
                ss = []
                for i in range(nslots):
                    sd, dd, rb, gsm, ssm = slots[i]
                    gs[i].wait()
                    ss.append(pltpu.async_copy(rb, acc.at[dd], ssm, add=True))
                for d_ in ss:
                    d_.wait()

            def flush_4(k, _2):
                flush_n(4 * k, 4)
                return 0
            lax.fori_loop(0, nb_c // 4, flush_4, 0)

            @pl.when(nb_c % 4 >= 2)
            def _():
                flush_n(nb_c - (nb_c % 4), 2)

            @pl.when(nb_c % 2 == 1)
            def _():
                flush_n(nb_c - 1, 1)
            return 0
        nch = (nbat + (MACRO // BATCH) - 1) // (MACRO // BATCH)
        lax.fori_loop(0, nch, chunk_step, 0)
        plsc.subcore_barrier()

        mych, start_row = _own_rows(s, block)

        def out_chunk(q, _):
            r0 = pl.multiple_of(start_row + q * 16, 16)
            pltpu.sync_copy(acc.at[pl.ds(r0, 16)], rowsbuf.at[pl.ds(0, 16)])
            pltpu.sync_copy(invin.at[pl.ds(base + r0, 16)],
                            cnt.at[pl.ds(pl.multiple_of(q * 16, 16), 16)])
            _scale_rows(q, cnt, rowsbuf)
            pltpu.sync_copy(rowsbuf.at[pl.ds(0, 16)],
                            out.at[pl.ds(base + r0, 16)])
            return 0
        lax.fori_loop(0, mych, out_chunk, 0)
        plsc.subcore_barrier()


_MESH = plsc.VectorSubcoreMesh(core_axis_name="c", subcore_axis_name="s",
                               num_cores=NC, num_subcores=NS)
_SEGTOT = NBLK * NS * SEGCAP


@jax.jit
def _smean_scan(table, srcp, dstp):
    f = pl.kernel(
        _scan_body,
        out_type=(
            jax.ShapeDtypeStruct((NROWS, D), jnp.float32),
            jax.ShapeDtypeStruct((_SEGTOT,), jnp.int32),
            jax.ShapeDtypeStruct((_SEGTOT,), jnp.int32),
            jax.ShapeDtypeStruct((NC * NS * L,), jnp.int32),
            jax.ShapeDtypeStruct((NROWS,), jnp.float32),
        ),
        mesh=_MESH,
        scratch_types=[
            pltpu.VMEM((MACRO,), jnp.int32),
            pltpu.VMEM((MACRO,), jnp.int32),
            pltpu.VMEM((MACRO + BATCH,), jnp.int32),
            pltpu.VMEM((MACRO + BATCH,), jnp.int32),
            pltpu.VMEM((BATCH,), jnp.int32),
            pltpu.VMEM((BATCH,), jnp.int32),
            pltpu.VMEM((BATCH,), jnp.int32),
            pltpu.VMEM((BATCH,), jnp.int32),
            pltpu.VMEM((BATCH,), jnp.int32),
            pltpu.VMEM((BATCH,), jnp.int32),
            pltpu.VMEM((BATCH,), jnp.int32),
            pltpu.VMEM((BATCH,), jnp.int32),
            pltpu.VMEM((BATCH, D), jnp.float32),
            pltpu.VMEM((BATCH, D), jnp.float32),
            pltpu.VMEM((BATCH, D), jnp.float32),
            pltpu.VMEM((BATCH, D), jnp.float32),
            pltpu.VMEM((DB,), jnp.float32),
            pltpu.VMEM((NS * CW,), jnp.float32),
            pltpu.VMEM((L,), jnp.int32),
            pltpu.VMEM_SHARED((ACC_ROWS, D), jnp.float32),
            pltpu.VMEM_SHARED((NS * DB,), jnp.float32),
        ] + [pltpu.SemaphoreType.DMA] * 10,
        compiler_params=pltpu.CompilerParams(needs_layout_passes=False),
    )
    return f(table, srcp, dstp)


@jax.jit
def _smean_replay(table, srcc, dlocc, counts, invin):
    f = pl.kernel(
        _replay_body,
        out_type=jax.ShapeDtypeStruct((NROWS, D), jnp.float32),
        mesh=_MESH,
        scratch_types=[
            pltpu.VMEM((MACRO,), jnp.int32),
            pltpu.VMEM((MACRO,), jnp.int32),
            pltpu.VMEM((BATCH,), jnp.int32),
            pltpu.VMEM((BATCH,), jnp.int32),
            pltpu.VMEM((BATCH,), jnp.int32),
            pltpu.VMEM((BATCH,), jnp.int32),
            pltpu.VMEM((BATCH,), jnp.int32),
            pltpu.VMEM((BATCH,), jnp.int32),
            pltpu.VMEM((BATCH,), jnp.int32),
            pltpu.VMEM((BATCH,), jnp.int32),
            pltpu.VMEM((BATCH, D), jnp.float32),
            pltpu.VMEM((BATCH, D), jnp.float32),
            pltpu.VMEM((BATCH, D), jnp.float32),
            pltpu.VMEM((BATCH, D), jnp.float32),
            pltpu.VMEM((DB,), jnp.float32),
            pltpu.VMEM((L,), jnp.int32),
            pltpu.VMEM_SHARED((ACC_ROWS, D), jnp.float32),
        ] + [pltpu.SemaphoreType.DMA] * 8,
        compiler_params=pltpu.CompilerParams(needs_layout_passes=False),
    )
    return f(table, srcc, dlocc, counts, invin)


def _pad_edges(e):
    src = e[0].astype(jnp.int32)
    dst = e[1].astype(jnp.int32)
    pad = EP - E
    srcp = jnp.concatenate([src, jnp.zeros((pad,), jnp.int32)])
    dstp = jnp.concatenate([dst, jnp.full((pad,), -1, jnp.int32)])
    return srcp, dstp


def kernel(x_users, x_artists, edge_index_a2u, edge_index_u2a):
    sa, da = _pad_edges(edge_index_a2u)
    su, du = _pad_edges(edge_index_u2a)
    xu, xa = x_users, x_artists
    xu, a_srcc, a_dlocc, a_counts, a_inv = _smean_scan(xa, sa, da)
    xa, u_srcc, u_dlocc, u_counts, u_inv = _smean_scan(xu, su, du)
    fu = x_users + xu
    fa = x_artists + xa
    for _ in range(2):
        xu = _smean_replay(xa, a_srcc, a_dlocc, a_counts, a_inv)
        xa = _smean_replay(xu, u_srcc, u_dlocc, u_counts, u_inv)
        fu = fu + xu
        fa = fa + xa
    return (0.25 * fu, 0.25 * fa)
